# Initial kernel scaffold; baseline (speedup 1.0000x reference)
import jax
import jax.numpy as jnp
from jax.experimental import pallas as pl


def kernel(hidden_states, weight):
    raise NotImplementedError("write your pallas kernel here")


# speedup vs baseline: 1.1486x; 1.1486x over previous
import functools

import jax
import jax.numpy as jnp
from jax.experimental import pallas as pl

_E = 64
_K = 8
_BT = 512


def _gate_kernel(x_ref, wt_ref, idx_ref, score_ref):
    x = x_ref[...]
    wt = wt_ref[...]
    logits = jnp.dot(x, wt, preferred_element_type=jnp.float32)
    m = jnp.max(logits, axis=-1, keepdims=True)
    e = jnp.exp(logits - m)
    scores = e / jnp.sum(e, axis=-1, keepdims=True)

    bt = scores.shape[0]
    iota = jax.lax.broadcasted_iota(jnp.int32, (bt, _E), 1)
    idx_cols = []
    val_cols = []
    work = scores
    for _ in range(_K):
        vmax = jnp.max(work, axis=-1, keepdims=True)
        cand = jnp.where(work == vmax, iota, _E)
        imin = jnp.min(cand, axis=-1, keepdims=True)
        idx_cols.append(imin)
        val_cols.append(vmax)
        work = jnp.where(iota == imin, -1.0, work)
    topk_idx = jnp.concatenate(idx_cols, axis=-1)
    topk_val = jnp.concatenate(val_cols, axis=-1)
    denom = jnp.sum(topk_val, axis=-1, keepdims=True) + 1e-20
    idx_ref[...] = topk_idx
    score_ref[...] = topk_val / denom


@functools.partial(jax.jit, static_argnames=())
def kernel(hidden_states, weight):
    b, s, d = hidden_states.shape
    t = b * s
    x = hidden_states.reshape(t, d)
    wt = weight.T
    grid = (t // _BT,)
    idx, scores = pl.pallas_call(
        _gate_kernel,
        grid=grid,
        in_specs=[
            pl.BlockSpec((_BT, d), lambda i: (i, 0)),
            pl.BlockSpec((d, _E), lambda i: (0, 0)),
        ],
        out_specs=[
            pl.BlockSpec((_BT, _K), lambda i: (i, 0)),
            pl.BlockSpec((_BT, _K), lambda i: (i, 0)),
        ],
        out_shape=[
            jax.ShapeDtypeStruct((t, _K), jnp.int32),
            jax.ShapeDtypeStruct((t, _K), jnp.float32),
        ],
    )(x, wt)
    aux_loss = jnp.zeros((), dtype=jnp.float32)
    return (idx, scores, aux_loss)


# speedup vs baseline: 1.7625x; 1.5345x over previous
import functools

import jax
import jax.numpy as jnp
from jax.experimental import pallas as pl

_E = 64
_K = 8
_BT = 512


def _gate_kernel(x_ref, w_ref, idx_ref, score_ref):
    x = x_ref[...]
    w = w_ref[...]
    logits = jax.lax.dot_general(
        w, x, (((1,), (1,)), ((), ())), preferred_element_type=jnp.float32
    )
    bt = logits.shape[1]
    iota = jax.lax.broadcasted_iota(jnp.int32, (_E, bt), 0)
    idx_rows = []
    val_rows = []
    work = logits
    for _ in range(_K):
        vmax = jnp.max(work, axis=0, keepdims=True)
        cand = jnp.where(work == vmax, iota, _E)
        imin = jnp.min(cand, axis=0, keepdims=True)
        idx_rows.append(imin)
        val_rows.append(vmax)
        work = jnp.where(iota == imin, -1e30, work)
    topk_idx = jnp.concatenate(idx_rows, axis=0)
    topk_logit = jnp.concatenate(val_rows, axis=0)
    e = jnp.exp(topk_logit - topk_logit[0:1])
    denom = jnp.sum(e, axis=0, keepdims=True)
    idx_ref[...] = topk_idx
    score_ref[...] = e / denom


@functools.partial(jax.jit, static_argnames=())
def kernel(hidden_states, weight):
    b, s, d = hidden_states.shape
    t = b * s
    x = hidden_states.reshape(t, d)
    grid = (t // _BT,)
    idx_t, scores_t = pl.pallas_call(
        _gate_kernel,
        grid=grid,
        in_specs=[
            pl.BlockSpec((_BT, d), lambda i: (i, 0)),
            pl.BlockSpec((_E, d), lambda i: (0, 0)),
        ],
        out_specs=[
            pl.BlockSpec((_K, _BT), lambda i: (0, i)),
            pl.BlockSpec((_K, _BT), lambda i: (0, i)),
        ],
        out_shape=[
            jax.ShapeDtypeStruct((_K, t), jnp.int32),
            jax.ShapeDtypeStruct((_K, t), jnp.float32),
        ],
    )(x, weight)
    aux_loss = jnp.zeros((), dtype=jnp.float32)
    return (idx_t.T, scores_t.T, aux_loss)


# speedup vs baseline: 1.8899x; 1.0723x over previous
import functools

import jax
import jax.numpy as jnp
from jax.experimental import pallas as pl

_E = 64
_K = 8
_BT = 1024


def _gate_kernel(x_ref, w_ref, idx_ref, score_ref):
    x = x_ref[...]
    w = w_ref[...]
    logits = jax.lax.dot_general(
        w, x, (((1,), (1,)), ((), ())), preferred_element_type=jnp.float32
    )
    bt = logits.shape[1]
    iota = jax.lax.broadcasted_iota(jnp.int32, (_E, bt), 0)
    idx_rows = []
    val_rows = []
    work = logits
    for _ in range(_K):
        vmax = jnp.max(work, axis=0, keepdims=True)
        cand = jnp.where(work == vmax, iota, _E)
        imin = jnp.min(cand, axis=0, keepdims=True)
        idx_rows.append(imin)
        val_rows.append(vmax)
        work = jnp.where(iota == imin, -1e30, work)
    topk_idx = jnp.concatenate(idx_rows, axis=0)
    topk_logit = jnp.concatenate(val_rows, axis=0)
    e = jnp.exp(topk_logit - topk_logit[0:1])
    denom = jnp.sum(e, axis=0, keepdims=True)
    idx_ref[...] = topk_idx
    score_ref[...] = e / denom


@functools.partial(jax.jit, static_argnames=())
def kernel(hidden_states, weight):
    b, s, d = hidden_states.shape
    t = b * s
    x = hidden_states.reshape(t, d)
    grid = (t // _BT,)
    idx_t, scores_t = pl.pallas_call(
        _gate_kernel,
        grid=grid,
        in_specs=[
            pl.BlockSpec((_BT, d), lambda i: (i, 0)),
            pl.BlockSpec((_E, d), lambda i: (0, 0)),
        ],
        out_specs=[
            pl.BlockSpec((_K, _BT), lambda i: (0, i)),
            pl.BlockSpec((_K, _BT), lambda i: (0, i)),
        ],
        out_shape=[
            jax.ShapeDtypeStruct((_K, t), jnp.int32),
            jax.ShapeDtypeStruct((_K, t), jnp.float32),
        ],
    )(x, weight)
    aux_loss = jnp.zeros((), dtype=jnp.float32)
    return (idx_t.T, scores_t.T, aux_loss)
